# baseline (speedup 1.0000x reference)
import jax
import jax.numpy as jnp
from jax.experimental import pallas as pl
from jax.experimental.pallas import tpu as pltpu

_B, _H, _W = 16, 512, 512
_N = _B * _H * _W
_P1 = 16
_BB2 = 2
_S2 = _B // _BB2


def _fused_body(x_hbm, m_ref, c_ref, bufx, emask, mbuf, s_ref, in_sems):
    s = pl.program_id(0)

    @pl.when(s == 0)
    def _prologue():
        s_ref[0, 0] = jnp.float32(0.0)
        mbuf[pl.ds(0, 1)] = jnp.full((1, _W), -jnp.inf, jnp.float32)
        mbuf[pl.ds(_H + 1, 1)] = jnp.full((1, _W), -jnp.inf, jnp.float32)
        for b in range(_P1):
            pltpu.make_async_copy(
                x_hbm.at[pl.ds(b, 1)],
                bufx.at[pl.ds(b, 1)],
                in_sems.at[b],
            ).start()

    @pl.when(s == 0)
    def _phase1():
        ninf = jnp.float32(-jnp.inf)
        col = jax.lax.broadcasted_iota(jnp.int32, (1, _H, _W), 2)
        row = jax.lax.broadcasted_iota(jnp.int32, (1, _H, _W), 1)

        def _one(b, acc):
            pltpu.make_async_copy(
                x_hbm.at[pl.ds(b, 1)],
                bufx.at[pl.ds(b, 1)],
                in_sems.at[b],
            ).wait()
            x = bufx[pl.ds(b, 1)]
            m = jnp.maximum(
                jnp.maximum(
                    jnp.where(col > 0, pltpu.roll(x, 1, 2), ninf),
                    jnp.where(col < _W - 1, pltpu.roll(x, _W - 1, 2), ninf),
                ),
                x,
            )
            mbuf[pl.ds(1, _H)] = m[0]
            pooled = jnp.maximum(
                jnp.maximum(mbuf[pl.ds(0, _H)], mbuf[pl.ds(1, _H)]),
                mbuf[pl.ds(2, _H)],
            )[None]
            emask[pl.ds(b, 1)] = x == pooled
            return acc + jnp.sum(x)

        s_ref[0, 0] = jax.lax.fori_loop(0, _P1, _one, jnp.float32(0.0))

    @pl.when(s >= 1)
    def _phase2():
        i = s - 1
        mean = s_ref[0, 0] * jnp.float32(1.0 / _N)
        x = bufx[pl.ds(i * _BB2, _BB2)]
        e = emask[pl.ds(i * _BB2, _BB2)]
        mask = e & (x > mean)
        m_ref[...] = mask

        @pl.when(s == 1)
        def _init_cnt():
            c_ref[0, 0] = jnp.int32(0)

        c_ref[0, 0] += jnp.sum(mask.astype(jnp.int32))


def kernel(input):
    x3 = input.reshape(_B, _H, _W)
    mask, cnt = pl.pallas_call(
        _fused_body,
        grid=(1 + _S2,),
        in_specs=[pl.BlockSpec(memory_space=pl.ANY)],
        out_specs=[
            pl.BlockSpec((_BB2, _H, _W), lambda s: (jnp.maximum(s - 1, 0), 0, 0)),
            pl.BlockSpec(memory_space=pltpu.SMEM),
        ],
        out_shape=[
            jax.ShapeDtypeStruct((_B, _H, _W), jnp.bool_),
            jax.ShapeDtypeStruct((1, 1), jnp.int32),
        ],
        scratch_shapes=[
            pltpu.VMEM((_B, _H, _W), jnp.float32),
            pltpu.VMEM((_B, _H, _W), jnp.bool_),
            pltpu.VMEM((_H + 2, _W), jnp.float32),
            pltpu.SMEM((1, 1), jnp.float32),
            pltpu.SemaphoreType.DMA((_P1,)),
        ],
    )(x3)
    return mask.reshape(_B, 1, _H, _W), cnt[0, 0]


# speedup vs baseline: 1.0562x; 1.0562x over previous
import jax
import jax.numpy as jnp
from jax.experimental import pallas as pl
from jax.experimental.pallas import tpu as pltpu

_B, _H, _W = 16, 512, 512
_N = _B * _H * _W
_BB = 4
_S = _B // _BB


def _fused_body(x_hbm, m_ref, c_ref, land, buf, s_ref, in_sems):
    s = pl.program_id(0)

    @pl.when(s == 0)
    def _prologue():
        s_ref[0, 0] = jnp.float32(0.0)
        pltpu.make_async_copy(
            x_hbm.at[pl.ds(0, _BB)], land.at[0], in_sems.at[0]
        ).start()
        pltpu.make_async_copy(
            x_hbm.at[pl.ds(_BB, _BB)], land.at[1], in_sems.at[1]
        ).start()

    @pl.when(s < _S)
    def _phase1():
        slot = jax.lax.rem(s, 2)
        pltpu.make_async_copy(
            x_hbm.at[pl.ds(s * _BB, _BB)], land.at[slot], in_sems.at[slot]
        ).wait()

        x = land[slot]
        ninf = jnp.float32(-jnp.inf)
        col = jax.lax.broadcasted_iota(jnp.int32, (_BB, _H, _W), 2)
        row = jax.lax.broadcasted_iota(jnp.int32, (_BB, _H, _W), 1)
        m = jnp.maximum(
            jnp.maximum(
                jnp.where(col > 0, pltpu.roll(x, 1, 2), ninf),
                jnp.where(col < _W - 1, pltpu.roll(x, _W - 1, 2), ninf),
            ),
            x,
        )
        pooled = jnp.maximum(
            jnp.maximum(
                jnp.where(row > 0, pltpu.roll(m, 1, 1), ninf),
                jnp.where(row < _H - 1, pltpu.roll(m, _H - 1, 1), ninf),
            ),
            m,
        )
        buf[pl.ds(s * _BB, _BB)] = jnp.where(x == pooled, x, ninf)
        s_ref[0, 0] += jnp.sum(x)

        @pl.when(s + 2 < _S)
        def _prefetch():
            pltpu.make_async_copy(
                x_hbm.at[pl.ds((s + 2) * _BB, _BB)],
                land.at[slot],
                in_sems.at[slot],
            ).start()

    @pl.when(s >= _S)
    def _phase2():
        i = s - _S
        mean = s_ref[0, 0] * jnp.float32(1.0 / _N)
        y = buf[pl.ds(i * _BB, _BB)]
        mask = y > mean
        m_ref[...] = mask

        @pl.when(s == _S)
        def _init_cnt():
            c_ref[0, 0] = jnp.int32(0)

        c_ref[0, 0] += jnp.sum(mask.astype(jnp.int32))


def kernel(input):
    x3 = input.reshape(_B, _H, _W)
    mask, cnt = pl.pallas_call(
        _fused_body,
        grid=(2 * _S,),
        in_specs=[pl.BlockSpec(memory_space=pl.ANY)],
        out_specs=[
            pl.BlockSpec((_BB, _H, _W), lambda s: (jnp.maximum(s - _S, 0), 0, 0)),
            pl.BlockSpec(memory_space=pltpu.SMEM),
        ],
        out_shape=[
            jax.ShapeDtypeStruct((_B, _H, _W), jnp.bool_),
            jax.ShapeDtypeStruct((1, 1), jnp.int32),
        ],
        scratch_shapes=[
            pltpu.VMEM((2, _BB, _H, _W), jnp.float32),
            pltpu.VMEM((_B, _H, _W), jnp.float32),
            pltpu.SMEM((1, 1), jnp.float32),
            pltpu.SemaphoreType.DMA((2,)),
        ],
    )(x3)
    return mask.reshape(_B, 1, _H, _W), cnt[0, 0]


# speedup vs baseline: 1.0940x; 1.0358x over previous
import jax
import jax.numpy as jnp
from jax.experimental import pallas as pl
from jax.experimental.pallas import tpu as pltpu

_B, _H, _W = 16, 512, 512
_N = _B * _H * _W
_BB = 4
_S = _B // _BB


def _fused_body(x_hbm, m_ref, c_ref, land, buf, s_ref, in_sems):
    s = pl.program_id(0)

    @pl.when(s == 0)
    def _prologue():
        s_ref[0, 0] = jnp.float32(0.0)
        pltpu.make_async_copy(
            x_hbm.at[pl.ds(0, _BB)], land.at[0], in_sems.at[0]
        ).start()
        pltpu.make_async_copy(
            x_hbm.at[pl.ds(_BB, _BB)], land.at[1], in_sems.at[1]
        ).start()

    @pl.when(s < _S)
    def _phase1():
        slot = jax.lax.rem(s, 2)
        pltpu.make_async_copy(
            x_hbm.at[pl.ds(s * _BB, _BB)], land.at[slot], in_sems.at[slot]
        ).wait()

        x = land[slot]
        ninf = jnp.float32(-jnp.inf)
        row = jax.lax.broadcasted_iota(jnp.int32, (_BB, _H, _W), 1)
        pad = jnp.full((_BB, _H, 1), ninf, jnp.float32)
        m = jnp.maximum(
            jnp.maximum(
                jnp.concatenate([x[:, :, 1:], pad], axis=2),
                jnp.concatenate([pad, x[:, :, : _W - 1]], axis=2),
            ),
            x,
        )
        pooled = jnp.maximum(
            jnp.maximum(
                jnp.where(row > 0, pltpu.roll(m, 1, 1), ninf),
                jnp.where(row < _H - 1, pltpu.roll(m, _H - 1, 1), ninf),
            ),
            m,
        )
        buf[pl.ds(s * _BB, _BB)] = jnp.where(x == pooled, x, ninf)
        s_ref[0, 0] += jnp.sum(x)

        @pl.when(s + 2 < _S)
        def _prefetch():
            pltpu.make_async_copy(
                x_hbm.at[pl.ds((s + 2) * _BB, _BB)],
                land.at[slot],
                in_sems.at[slot],
            ).start()

    @pl.when(s >= _S)
    def _phase2():
        i = s - _S
        mean = s_ref[0, 0] * jnp.float32(1.0 / _N)
        y = buf[pl.ds(i * _BB, _BB)]
        mask = y > mean
        m_ref[...] = mask

        @pl.when(s == _S)
        def _init_cnt():
            c_ref[0, 0] = jnp.int32(0)

        c_ref[0, 0] += jnp.sum(mask.astype(jnp.int32))


def kernel(input):
    x3 = input.reshape(_B, _H, _W)
    mask, cnt = pl.pallas_call(
        _fused_body,
        grid=(2 * _S,),
        in_specs=[pl.BlockSpec(memory_space=pl.ANY)],
        out_specs=[
            pl.BlockSpec((_BB, _H, _W), lambda s: (jnp.maximum(s - _S, 0), 0, 0)),
            pl.BlockSpec(memory_space=pltpu.SMEM),
        ],
        out_shape=[
            jax.ShapeDtypeStruct((_B, _H, _W), jnp.bool_),
            jax.ShapeDtypeStruct((1, 1), jnp.int32),
        ],
        scratch_shapes=[
            pltpu.VMEM((2, _BB, _H, _W), jnp.float32),
            pltpu.VMEM((_B, _H, _W), jnp.float32),
            pltpu.SMEM((1, 1), jnp.float32),
            pltpu.SemaphoreType.DMA((2,)),
        ],
    )(x3)
    return mask.reshape(_B, 1, _H, _W), cnt[0, 0]


# speedup vs baseline: 1.0986x; 1.0042x over previous
import jax
import jax.numpy as jnp
from jax.experimental import pallas as pl
from jax.experimental.pallas import tpu as pltpu

_B, _H, _W = 16, 512, 512
_N = _B * _H * _W
_BB = 4
_S = _B // _BB


def _fused_body(x_hbm, m_ref, c_ref, land, buf, s_ref, in_sems):
    s = pl.program_id(0)

    @pl.when(s == 0)
    def _prologue():
        s_ref[0, 0] = jnp.float32(0.0)
        pltpu.make_async_copy(
            x_hbm.at[pl.ds(0, _BB)], land.at[0], in_sems.at[0]
        ).start()
        pltpu.make_async_copy(
            x_hbm.at[pl.ds(_BB, _BB)], land.at[1], in_sems.at[1]
        ).start()

    @pl.when(s < _S)
    def _phase1():
        slot = jax.lax.rem(s, 2)
        pltpu.make_async_copy(
            x_hbm.at[pl.ds(s * _BB, _BB)], land.at[slot], in_sems.at[slot]
        ).wait()

        x = land[slot]
        ninf = jnp.float32(-jnp.inf)
        pad = jnp.full((_BB, _H, 1), ninf, jnp.float32)
        padrow = jnp.full((_BB, 1, _W), ninf, jnp.float32)
        m = jnp.maximum(
            jnp.maximum(
                jnp.concatenate([x[:, :, 1:], pad], axis=2),
                jnp.concatenate([pad, x[:, :, : _W - 1]], axis=2),
            ),
            x,
        )
        pooled = jnp.maximum(
            jnp.maximum(
                jnp.concatenate([m[:, 1:, :], padrow], axis=1),
                jnp.concatenate([padrow, m[:, : _H - 1, :]], axis=1),
            ),
            m,
        )
        buf[pl.ds(s * _BB, _BB)] = jnp.where(x == pooled, x, ninf)
        s_ref[0, 0] += jnp.sum(x)

        @pl.when(s + 2 < _S)
        def _prefetch():
            pltpu.make_async_copy(
                x_hbm.at[pl.ds((s + 2) * _BB, _BB)],
                land.at[slot],
                in_sems.at[slot],
            ).start()

    @pl.when(s >= _S)
    def _phase2():
        i = s - _S
        mean = s_ref[0, 0] * jnp.float32(1.0 / _N)
        y = buf[pl.ds(i * _BB, _BB)]
        mask = y > mean
        m_ref[...] = mask

        @pl.when(s == _S)
        def _init_cnt():
            c_ref[0, 0] = jnp.int32(0)

        c_ref[0, 0] += jnp.sum(mask.astype(jnp.int32))


def kernel(input):
    x3 = input.reshape(_B, _H, _W)
    mask, cnt = pl.pallas_call(
        _fused_body,
        grid=(2 * _S,),
        in_specs=[pl.BlockSpec(memory_space=pl.ANY)],
        out_specs=[
            pl.BlockSpec((_BB, _H, _W), lambda s: (jnp.maximum(s - _S, 0), 0, 0)),
            pl.BlockSpec(memory_space=pltpu.SMEM),
        ],
        out_shape=[
            jax.ShapeDtypeStruct((_B, _H, _W), jnp.bool_),
            jax.ShapeDtypeStruct((1, 1), jnp.int32),
        ],
        scratch_shapes=[
            pltpu.VMEM((2, _BB, _H, _W), jnp.float32),
            pltpu.VMEM((_B, _H, _W), jnp.float32),
            pltpu.SMEM((1, 1), jnp.float32),
            pltpu.SemaphoreType.DMA((2,)),
        ],
    )(x3)
    return mask.reshape(_B, 1, _H, _W), cnt[0, 0]
